# baseline (speedup 1.0000x reference)
import functools

import jax
import jax.numpy as jnp
from jax import lax
from jax.experimental import pallas as pl
from jax.experimental.pallas import tpu as pltpu
from jax.experimental.pallas import tpu_sc as plsc


def _make_sc_gather(n_rows: int, emb: int, num_cores: int, num_subcores: int):
    nw = num_cores * num_subcores
    rows_per_w = max(8, n_rows // nw)
    n_active = n_rows // rows_per_w
    mesh = plsc.VectorSubcoreMesh(core_axis_name="c", subcore_axis_name="s")

    @functools.partial(
        pl.kernel,
        mesh=mesh,
        out_type=jax.ShapeDtypeStruct((n_rows, emb), jnp.float32),
        scratch_types=[
            pltpu.VMEM((rows_per_w,), jnp.int32),
            pltpu.VMEM((rows_per_w, emb), jnp.float32),
            pltpu.SemaphoreType.DMA,
        ],
        compiler_params=pltpu.CompilerParams(use_tc_tiling_on_sc=False),
    )
    def gather(idx_hbm, table_hbm, pe_hbm, idx_v, rows_v, sem):
        wid = lax.axis_index("s") * num_cores + lax.axis_index("c")

        @pl.when(wid < n_active)
        def _():
            base = wid * rows_per_w
            pltpu.sync_copy(idx_hbm.at[pl.ds(base, rows_per_w)], idx_v)
            pltpu.async_copy(table_hbm.at[idx_v], rows_v, sem).wait()
            pltpu.sync_copy(rows_v, pe_hbm.at[pl.ds(base, rows_per_w)])

    return gather


def _add_body(x_ref, pe_ref, o_ref):
    o_ref[...] = x_ref[...] + pe_ref[...][None, :, :]


def kernel(x, input_channels, pos_embed):
    B, N, P, E = x.shape
    input_channels = input_channels.astype(jnp.int32)

    info = plsc.get_sparse_core_info()
    gather = _make_sc_gather(N, E, info.num_cores, info.num_subcores)
    pe = gather(input_channels, pos_embed)

    xt = jnp.transpose(x, (0, 2, 3, 1)).reshape(B * P, E, N)
    pe_t = pe.T

    CH = 50
    out_t = pl.pallas_call(
        _add_body,
        grid=(B * P // CH,),
        in_specs=[
            pl.BlockSpec((CH, E, N), lambda c: (c, 0, 0)),
            pl.BlockSpec((E, N), lambda c: (0, 0)),
        ],
        out_specs=pl.BlockSpec((CH, E, N), lambda c: (c, 0, 0)),
        out_shape=jax.ShapeDtypeStruct((B * P, E, N), jnp.float32),
    )(xt, pe_t)
    return jnp.transpose(out_t.reshape(B, P, E, N), (0, 3, 1, 2))


# speedup vs baseline: 1.1747x; 1.1747x over previous
import functools

import jax
import jax.numpy as jnp
from jax import lax
from jax.experimental import pallas as pl
from jax.experimental.pallas import tpu as pltpu
from jax.experimental.pallas import tpu_sc as plsc


def _make_sc_gather(n_rows: int, emb: int, num_cores: int, num_subcores: int):
    nw = num_cores * num_subcores
    rows_per_w = max(8, n_rows // nw)
    n_active = n_rows // rows_per_w
    mesh = plsc.VectorSubcoreMesh(core_axis_name="c", subcore_axis_name="s")

    @functools.partial(
        pl.kernel,
        mesh=mesh,
        out_type=jax.ShapeDtypeStruct((n_rows, emb), jnp.float32),
        scratch_types=[
            pltpu.VMEM((rows_per_w,), jnp.int32),
            pltpu.VMEM((rows_per_w, emb), jnp.float32),
            pltpu.SemaphoreType.DMA,
        ],
        compiler_params=pltpu.CompilerParams(use_tc_tiling_on_sc=False),
    )
    def gather(idx_hbm, table_hbm, pe_hbm, idx_v, rows_v, sem):
        wid = lax.axis_index("s") * num_cores + lax.axis_index("c")

        @pl.when(wid < n_active)
        def _():
            base = wid * rows_per_w
            pltpu.sync_copy(idx_hbm.at[pl.ds(base, rows_per_w)], idx_v)
            pltpu.async_copy(table_hbm.at[idx_v], rows_v, sem).wait()
            pltpu.sync_copy(rows_v, pe_hbm.at[pl.ds(base, rows_per_w)])

    return gather


def _add_body(x_ref, pe_ref, o_ref):
    o_ref[...] = x_ref[...] + pe_ref[...][None, :, :]


def kernel(x, input_channels, pos_embed):
    B, N, P, E = x.shape
    input_channels = input_channels.astype(jnp.int32)

    pe = jnp.take(pos_embed, input_channels, axis=0)

    xt = jnp.transpose(x, (0, 2, 3, 1)).reshape(B * P, E, N)
    pe_t = pe.T

    CH = 50
    out_t = pl.pallas_call(
        _add_body,
        grid=(B * P // CH,),
        in_specs=[
            pl.BlockSpec((CH, E, N), lambda c: (c, 0, 0)),
            pl.BlockSpec((E, N), lambda c: (0, 0)),
        ],
        out_specs=pl.BlockSpec((CH, E, N), lambda c: (c, 0, 0)),
        out_shape=jax.ShapeDtypeStruct((B * P, E, N), jnp.float32),
    )(xt, pe_t)
    return jnp.transpose(out_t.reshape(B, P, E, N), (0, 3, 1, 2))


# speedup vs baseline: 1.2148x; 1.0341x over previous
import jax
import jax.numpy as jnp
from jax import lax
from jax.experimental import pallas as pl
from jax.experimental.pallas import tpu as pltpu


def _make_body(v: int, n: int):
    def body(idx_ref, tab_ref, x_ref, o_ref, pet_ref):
        @pl.when(pl.program_id(0) == 0)
        def _():
            idx = idx_ref[0, :]
            iota = lax.broadcasted_iota(jnp.int32, (v, n), 0)
            oh = jnp.where(iota == idx[None, :], 1.0, 0.0)
            pet_ref[...] = jnp.dot(tab_ref[...], oh,
                                   preferred_element_type=jnp.float32)

        o_ref[...] = x_ref[...] + pet_ref[...][None, :, :]

    return body


def kernel(x, input_channels, pos_embed):
    B, N, P, E = x.shape
    V = pos_embed.shape[0]
    idx2 = input_channels.astype(jnp.int32).reshape(1, N)
    tab_t = pos_embed.T
    xt = jnp.transpose(x, (0, 2, 3, 1)).reshape(B * P, E, N)

    CH = 50
    out_t = pl.pallas_call(
        _make_body(V, N),
        grid=(B * P // CH,),
        in_specs=[
            pl.BlockSpec((1, N), lambda c: (0, 0)),
            pl.BlockSpec((E, V), lambda c: (0, 0)),
            pl.BlockSpec((CH, E, N), lambda c: (c, 0, 0)),
        ],
        out_specs=pl.BlockSpec((CH, E, N), lambda c: (c, 0, 0)),
        out_shape=jax.ShapeDtypeStruct((B * P, E, N), jnp.float32),
        scratch_shapes=[pltpu.VMEM((E, N), jnp.float32)],
    )(idx2, tab_t, xt)
    return jnp.transpose(out_t.reshape(B, P, E, N), (0, 3, 1, 2))
